# baseline (speedup 1.0000x reference)
import jax
import jax.numpy as jnp
from jax.experimental import pallas as pl

_VOCAB = 156
_DIM = 19


def _cosine_top1_kernel(x_ref, w_ref, cos_ref, idx_ref):
    bB, L, D = x_ref.shape
    x = x_ref[...].reshape(bB * L, D)
    w = w_ref[...]
    dot = jax.lax.dot_general(
        x, w, (((1,), (1,)), ((), ())), preferred_element_type=jnp.float32
    )
    norm_e = jnp.sqrt(jnp.sum(x * x, axis=1, keepdims=True))
    norm_w = jnp.sqrt(jnp.sum(w * w, axis=1, keepdims=True))
    cos = dot / jnp.maximum(norm_e * norm_w.reshape(1, _VOCAB), 1e-8)
    cos_ref[...] = cos.reshape(bB, L, _VOCAB)
    idx = jnp.argmax(cos, axis=1).astype(jnp.int32)
    idx_ref[...] = idx.reshape(bB, L, 1)


def kernel(embedded_sequence, weight):
    B, L, D = embedded_sequence.shape
    bB = 16
    cos, idx = pl.pallas_call(
        _cosine_top1_kernel,
        grid=(B // bB,),
        in_specs=[
            pl.BlockSpec((bB, L, D), lambda i: (i, 0, 0)),
            pl.BlockSpec((_VOCAB, _DIM), lambda i: (0, 0)),
        ],
        out_specs=[
            pl.BlockSpec((bB, L, _VOCAB), lambda i: (i, 0, 0)),
            pl.BlockSpec((bB, L, 1), lambda i: (i, 0, 0)),
        ],
        out_shape=[
            jax.ShapeDtypeStruct((B, L, _VOCAB), jnp.float32),
            jax.ShapeDtypeStruct((B, L, 1), jnp.int32),
        ],
    )(embedded_sequence, weight)
    return cos, idx


# speedup vs baseline: 1.0351x; 1.0351x over previous
import jax
import jax.numpy as jnp
from jax.experimental import pallas as pl

_VOCAB = 156
_DIM = 19


def _cosine_top1_kernel(x_ref, w_ref, cos_ref, idx_ref):
    bB, L, D = x_ref.shape
    x = x_ref[...].reshape(bB * L, D)
    w = w_ref[...]
    dot = jax.lax.dot_general(
        x, w, (((1,), (1,)), ((), ())), preferred_element_type=jnp.float32
    )
    norm_e = jnp.sqrt(jnp.sum(x * x, axis=1, keepdims=True))
    norm_w = jnp.sqrt(jnp.sum(w * w, axis=1, keepdims=True))
    cos = dot / jnp.maximum(norm_e * norm_w.reshape(1, _VOCAB), 1e-8)
    cos_ref[...] = cos.reshape(bB, L, _VOCAB)
    idx_ref[...] = jnp.argmax(cos, axis=1, keepdims=True).astype(jnp.int32)


def kernel(embedded_sequence, weight):
    B, L, D = embedded_sequence.shape
    bB = 16
    cos, idx = pl.pallas_call(
        _cosine_top1_kernel,
        grid=(B // bB,),
        in_specs=[
            pl.BlockSpec((bB, L, D), lambda i: (i, 0, 0)),
            pl.BlockSpec((_VOCAB, _DIM), lambda i: (0, 0)),
        ],
        out_specs=[
            pl.BlockSpec((bB, L, _VOCAB), lambda i: (i, 0, 0)),
            pl.BlockSpec((bB * L, 1), lambda i: (i, 0)),
        ],
        out_shape=[
            jax.ShapeDtypeStruct((B, L, _VOCAB), jnp.float32),
            jax.ShapeDtypeStruct((B * L, 1), jnp.int32),
        ],
    )(embedded_sequence, weight)
    return cos, idx.reshape(B, L, 1)


# speedup vs baseline: 1.2521x; 1.2096x over previous
import jax
import jax.numpy as jnp
from jax.experimental import pallas as pl

_VOCAB = 156
_DIM = 19


def _cosine_top1_kernel(x_ref, w_ref, cos_ref, idx_ref):
    bB, L, D = x_ref.shape
    R = bB * L
    x = x_ref[...].reshape(R, D)
    w = w_ref[...]
    ones = jnp.ones((1, D), jnp.float32)
    dot = jax.lax.dot_general(
        x, w, (((1,), (1,)), ((), ())), preferred_element_type=jnp.float32
    )
    ne2 = jax.lax.dot_general(
        x * x, ones, (((1,), (1,)), ((), ())), preferred_element_type=jnp.float32
    )
    nw2 = jax.lax.dot_general(
        w * w, ones, (((1,), (1,)), ((), ())), preferred_element_type=jnp.float32
    )
    norm_e = jnp.sqrt(ne2)
    norm_w = jnp.sqrt(nw2)
    cos = dot / jnp.maximum(norm_e * norm_w.reshape(1, _VOCAB), 1e-8)
    cos_ref[...] = cos.reshape(bB, L, _VOCAB)
    dot_t = jax.lax.dot_general(
        w, x, (((1,), (1,)), ((), ())), preferred_element_type=jnp.float32
    )
    key = dot_t * (1.0 / norm_w)
    m = jnp.max(key, axis=0, keepdims=True)
    iota = jax.lax.broadcasted_iota(jnp.int32, (_VOCAB, R), 0)
    cand = jnp.where(key == m, iota, _VOCAB)
    idx = jnp.min(cand, axis=0)
    idx_ref[...] = idx.reshape(R, 1)


def kernel(embedded_sequence, weight):
    B, L, D = embedded_sequence.shape
    bB = 16
    cos, idx = pl.pallas_call(
        _cosine_top1_kernel,
        grid=(B // bB,),
        in_specs=[
            pl.BlockSpec((bB, L, D), lambda i: (i, 0, 0)),
            pl.BlockSpec((_VOCAB, _DIM), lambda i: (0, 0)),
        ],
        out_specs=[
            pl.BlockSpec((bB, L, _VOCAB), lambda i: (i, 0, 0)),
            pl.BlockSpec((bB * L, 1), lambda i: (i, 0)),
        ],
        out_shape=[
            jax.ShapeDtypeStruct((B, L, _VOCAB), jnp.float32),
            jax.ShapeDtypeStruct((B * L, 1), jnp.int32),
        ],
    )(embedded_sequence, weight)
    return cos, idx.reshape(B, L, 1)


# speedup vs baseline: 1.4329x; 1.1444x over previous
import jax
import jax.numpy as jnp
from jax.experimental import pallas as pl

_VOCAB = 156
_DIM = 19


def _cosine_top1_kernel(x_ref, w_ref, cos_ref, idx_ref):
    bB, L, D = x_ref.shape
    R = bB * L
    x = x_ref[...].reshape(R, D)
    w = w_ref[...]
    ones = jnp.ones((1, D), jnp.float32)
    dot = jax.lax.dot_general(
        x, w, (((1,), (1,)), ((), ())), preferred_element_type=jnp.float32
    )
    ne2 = jax.lax.dot_general(
        x * x, ones, (((1,), (1,)), ((), ())), preferred_element_type=jnp.float32
    )
    nw2 = jax.lax.dot_general(
        w * w, ones, (((1,), (1,)), ((), ())), preferred_element_type=jnp.float32
    )
    norm_e = jnp.sqrt(ne2)
    norm_w = jnp.sqrt(nw2)
    cos = dot / jnp.maximum(norm_e * norm_w.reshape(1, _VOCAB), 1e-8)
    cos_ref[...] = cos.reshape(bB, L, _VOCAB)
    dot_t = jax.lax.dot_general(
        w, x, (((1,), (1,)), ((), ())), preferred_element_type=jnp.float32
    )
    key = dot_t * (1.0 / norm_w)
    m = jnp.max(key, axis=0, keepdims=True)
    iota = jax.lax.broadcasted_iota(jnp.int32, (_VOCAB, R), 0)
    cand = jnp.where(key == m, iota, _VOCAB)
    idx = jnp.min(cand, axis=0, keepdims=True)
    idx_ref[...] = idx.reshape(1, 1, R)


def kernel(embedded_sequence, weight):
    B, L, D = embedded_sequence.shape
    bB = 16
    cos, idx = pl.pallas_call(
        _cosine_top1_kernel,
        grid=(B // bB,),
        in_specs=[
            pl.BlockSpec((bB, L, D), lambda i: (i, 0, 0)),
            pl.BlockSpec((_VOCAB, _DIM), lambda i: (0, 0)),
        ],
        out_specs=[
            pl.BlockSpec((bB, L, _VOCAB), lambda i: (i, 0, 0)),
            pl.BlockSpec((1, 1, bB * L), lambda i: (i, 0, 0)),
        ],
        out_shape=[
            jax.ShapeDtypeStruct((B, L, _VOCAB), jnp.float32),
            jax.ShapeDtypeStruct((B // bB, 1, bB * L), jnp.int32),
        ],
    )(embedded_sequence, weight)
    return cos, idx.reshape(B, L, 1)


# speedup vs baseline: 1.4912x; 1.0407x over previous
import jax
import jax.numpy as jnp
from jax.experimental import pallas as pl

_VOCAB = 156
_DIM = 19


def _cosine_top1_kernel(x_ref, w_ref, cos_ref, idx_ref):
    bB, L, D = x_ref.shape
    R = bB * L
    x = x_ref[...].reshape(R, D)
    w = w_ref[...]
    ones = jnp.ones((1, D), jnp.float32)
    dot = jax.lax.dot_general(
        x, w, (((1,), (1,)), ((), ())), preferred_element_type=jnp.float32
    )
    ne2 = jax.lax.dot_general(
        x * x, ones, (((1,), (1,)), ((), ())), preferred_element_type=jnp.float32
    )
    nw2 = jax.lax.dot_general(
        w * w, ones, (((1,), (1,)), ((), ())), preferred_element_type=jnp.float32
    )
    norm_e = jnp.sqrt(ne2)
    norm_w = jnp.sqrt(nw2)
    cos = dot / jnp.maximum(norm_e * norm_w.reshape(1, _VOCAB), 1e-8)
    cos_ref[...] = cos.reshape(bB, L, _VOCAB)
    dot_t = jax.lax.dot_general(
        w, x, (((1,), (1,)), ((), ())), preferred_element_type=jnp.float32
    )
    key = dot_t * (1.0 / norm_w)
    m = jnp.max(key, axis=0, keepdims=True)
    iota = jax.lax.broadcasted_iota(jnp.int32, (_VOCAB, R), 0)
    cand = jnp.where(key == m, iota, _VOCAB)
    idx = jnp.min(cand, axis=0, keepdims=True)
    idx_ref[...] = idx.reshape(1, 1, R)


def kernel(embedded_sequence, weight):
    B, L, D = embedded_sequence.shape
    bB = 32
    cos, idx = pl.pallas_call(
        _cosine_top1_kernel,
        grid=(B // bB,),
        in_specs=[
            pl.BlockSpec((bB, L, D), lambda i: (i, 0, 0)),
            pl.BlockSpec((_VOCAB, _DIM), lambda i: (0, 0)),
        ],
        out_specs=[
            pl.BlockSpec((bB, L, _VOCAB), lambda i: (i, 0, 0)),
            pl.BlockSpec((1, 1, bB * L), lambda i: (i, 0, 0)),
        ],
        out_shape=[
            jax.ShapeDtypeStruct((B, L, _VOCAB), jnp.float32),
            jax.ShapeDtypeStruct((B // bB, 1, bB * L), jnp.int32),
        ],
    )(embedded_sequence, weight)
    return cos, idx.reshape(B, L, 1)
